# baseline (speedup 1.0000x reference)
import functools

import jax
import jax.numpy as jnp
from jax import lax
from jax.experimental import pallas as pl
from jax.experimental.pallas import tpu as pltpu
from jax.experimental.pallas import tpu_sc as plsc

N = 10000
E = 160000
Q = 2512
NPAD = 4 * Q
CH = 128
CPT = 80
NCHUNK = 16 * CPT
EPAD = NCHUNK * CH
ROWS_Q = 2560
DUMMY0 = 2512
QSTRIPE = ROWS_Q // 16
CPW = NCHUNK // 32
LC = 48


def _z_body(x_ref, w_ref, dst_ref, xw_ref, dloc_ref):
    i = pl.program_id(0)
    xw_ref[...] = jnp.dot(x_ref[...], w_ref[...],
                          preferred_element_type=jnp.float32)

    @pl.when(i == 0)
    def _():
        d = dst_ref[...]
        dummy = DUMMY0 + (d & 31)
        dl = []
        for q in range(4):
            dq = d - q * Q
            dl.append(jnp.where((d >= q * Q) & (dq < Q), dq, dummy))
        dloc_ref[...] = jnp.stack(dl)


def _tc_z(x, W1, dst2d):
    return pl.pallas_call(
        _z_body,
        grid=(10,),
        in_specs=[
            pl.BlockSpec((1000, 256), lambda i: (i, 0)),
            pl.BlockSpec((256, 256), lambda i: (0, 0)),
            pl.BlockSpec((NCHUNK, CH), lambda i: (0, 0)),
        ],
        out_specs=[
            pl.BlockSpec((1000, 256), lambda i: (i, 0)),
            pl.BlockSpec((4, NCHUNK, CH), lambda i: (0, 0, 0)),
        ],
        out_shape=[
            jax.ShapeDtypeStruct((N, 256), jnp.float32),
            jax.ShapeDtypeStruct((4, NCHUNK, CH), jnp.int32),
        ],
    )(x, W1, dst2d)


def _compact_body(src_hbm, dloc_hbm, lsrc_hbm, ldst_hbm, cnt_hbm,
                  sstage, dstage, lsrc_v, ldst_v, cbuf):
    c = lax.axis_index("c")
    s = lax.axis_index("s")
    w = s * 2 + c
    pltpu.sync_copy(src_hbm.at[pl.ds(w * CPW, CPW)], sstage)
    lanes = lax.iota(jnp.int32, 16)

    for h in range(4):
        pltpu.sync_copy(dloc_hbm.at[h, pl.ds(w * CPW, CPW)], dstage)

        def chunk(k, off):
            for j in range(CH // 16):
                vd = dstage[k, pl.ds(j * 16, 16)]
                vs = sstage[k, pl.ds(j * 16, 16)]
                m = vd < Q
                mi = jnp.where(m, 1, 0)
                pos = off + plsc.cumsum(mi) - 1
                plsc.store_scatter(lsrc_v, [pos >> 7, pos & 127], vs, mask=m)
                plsc.store_scatter(ldst_v, [pos >> 7, pos & 127], vd, mask=m)
                off = off + jnp.sum(mi)
            return off

        off = lax.fori_loop(0, CPW, chunk, 0)
        for j in range(CH // 16):
            posp = off + j * 16 + lanes
            plsc.store_scatter(lsrc_v, [posp >> 7, posp & 127],
                               jnp.zeros((16,), jnp.int32))
            plsc.store_scatter(ldst_v, [posp >> 7, posp & 127],
                               DUMMY0 + (lanes & 15) + (j % 2) * 16)
        nch = (off + CH - 1) // CH
        cbuf[0, pl.ds(0, 16)] = jnp.full((16,), nch, jnp.int32)
        pltpu.sync_copy(lsrc_v, lsrc_hbm.at[h, w])
        pltpu.sync_copy(ldst_v, ldst_hbm.at[h, w])
        pltpu.sync_copy(cbuf, cnt_hbm.at[h, w])


def _sc_compact(src2d, dloc2d):
    kfn = functools.partial(
        pl.kernel,
        out_type=[
            jax.ShapeDtypeStruct((4, 32, LC, CH), jnp.int32),
            jax.ShapeDtypeStruct((4, 32, LC, CH), jnp.int32),
            jax.ShapeDtypeStruct((4, 32, 8, 128), jnp.int32),
        ],
        mesh=plsc.VectorSubcoreMesh(core_axis_name="c", subcore_axis_name="s"),
        compiler_params=pltpu.CompilerParams(needs_layout_passes=False),
        scratch_types=[
            pltpu.VMEM((CPW, CH), jnp.int32),
            pltpu.VMEM((CPW, CH), jnp.int32),
            pltpu.VMEM((LC, CH), jnp.int32),
            pltpu.VMEM((LC, CH), jnp.int32),
            pltpu.VMEM((8, 128), jnp.int32),
        ],
    )
    return kfn(_compact_body)(src2d, dloc2d)


def _deg_body(ldst_hbm, cnt_hbm, ones_hbm, zrows_hbm, deg_hbm,
              dlocb, cbuf, ones_v, sem, acc0, acc1):
    c = lax.axis_index("c")
    s = lax.axis_index("s")
    accs = (acc0, acc1)
    for qi in range(2):
        pltpu.sync_copy(zrows_hbm, accs[qi].at[pl.ds(s * QSTRIPE, QSTRIPE)])
    pltpu.sync_copy(ones_hbm, ones_v)
    plsc.subcore_barrier()

    for qi in range(2):
        acc = accs[qi]
        for w in (2 * s, 2 * s + 1):
            pltpu.sync_copy(ldst_hbm.at[2 * c + qi, w], dlocb)
            pltpu.sync_copy(cnt_hbm.at[2 * c + qi, w], cbuf)
            nch = jnp.max(cbuf[0, pl.ds(0, 16)])

            def chunk(k, carry):
                pltpu.async_copy(ones_v, acc.at[dlocb.at[k]], sem, add=True)
                return carry

            lax.fori_loop(0, nch, chunk, 0)

            def drain(k, carry):
                pltpu.make_async_copy(ones_v, acc.at[dlocb.at[0]], sem).wait()
                return carry

            lax.fori_loop(0, nch, drain, 0)

    plsc.subcore_barrier()
    for qi in range(2):
        @pl.when(s < 15)
        def _(qi=qi):
            pltpu.sync_copy(
                accs[qi].at[pl.ds(s * QSTRIPE, QSTRIPE)],
                deg_hbm.at[pl.ds((2 * c + qi) * Q + s * QSTRIPE, QSTRIPE)])

        @pl.when(s == 15)
        def _(qi=qi):
            pltpu.sync_copy(
                accs[qi].at[pl.ds(15 * QSTRIPE, Q - 15 * QSTRIPE)],
                deg_hbm.at[pl.ds((2 * c + qi) * Q + 15 * QSTRIPE,
                                 Q - 15 * QSTRIPE)])


def _sc_deg(ldst, cnts, ones128, zrows):
    kfn = functools.partial(
        pl.kernel,
        out_type=jax.ShapeDtypeStruct((NPAD, 128), jnp.float32),
        mesh=plsc.VectorSubcoreMesh(core_axis_name="c", subcore_axis_name="s"),
        compiler_params=pltpu.CompilerParams(needs_layout_passes=False),
        scratch_types=[
            pltpu.VMEM((LC, CH), jnp.int32),
            pltpu.VMEM((8, 128), jnp.int32),
            pltpu.VMEM((CH, 128), jnp.float32),
            pltpu.SemaphoreType.DMA,
            pltpu.VMEM_SHARED((ROWS_Q, 128), jnp.float32),
            pltpu.VMEM_SHARED((ROWS_Q, 128), jnp.float32),
        ],
    )
    return kfn(_deg_body)(ldst, cnts, ones128, zrows)


def _agg_body(p, lsrc_hbm, ldst_hbm, cnt_hbm, zrows_hbm, y_hbm, out_hbm,
              srcb, dlocb, cbuf, rows0, rows1, g0, g1, acc):
    c = lax.axis_index("c")
    s = lax.axis_index("s")
    q = 2 * p + c
    pltpu.sync_copy(zrows_hbm, acc.at[pl.ds(s * QSTRIPE, QSTRIPE)])
    plsc.subcore_barrier()

    bufs = (rows0, rows1)
    gsems = (g0, g1)

    def start_gather(k, b):
        pltpu.make_async_copy(y_hbm.at[srcb.at[k]], bufs[b], gsems[b]).start()

    def wait_gather(b):
        pltpu.make_async_copy(y_hbm.at[srcb.at[0]], bufs[b], gsems[b]).wait()

    def scatter(k, b):
        pltpu.sync_copy(bufs[b], acc.at[dlocb.at[k]], add=True)

    for w in (2 * s, 2 * s + 1):
        pltpu.sync_copy(lsrc_hbm.at[q, w], srcb)
        pltpu.sync_copy(ldst_hbm.at[q, w], dlocb)
        pltpu.sync_copy(cnt_hbm.at[q, w], cbuf)
        nch = jnp.max(cbuf[0, pl.ds(0, 16)])

        @pl.when(nch > 0)
        def _():
            start_gather(0, 0)

        def pair(k2, carry):
            k0 = 2 * k2
            k1 = k0 + 1

            @pl.when(k1 < nch)
            def _():
                start_gather(k1, 1)

            wait_gather(0)
            scatter(k0, 0)

            @pl.when(k0 + 2 < nch)
            def _():
                start_gather(k0 + 2, 0)

            @pl.when(k1 < nch)
            def _():
                wait_gather(1)
                scatter(k1, 1)

            return carry

        lax.fori_loop(0, (nch + 1) // 2, pair, 0)

    plsc.subcore_barrier()

    @pl.when(s < 15)
    def _():
        pltpu.sync_copy(acc.at[pl.ds(s * QSTRIPE, QSTRIPE)],
                        out_hbm.at[pl.ds(c * Q + s * QSTRIPE, QSTRIPE)])

    @pl.when(s == 15)
    def _():
        pltpu.sync_copy(acc.at[pl.ds(15 * QSTRIPE, Q - 15 * QSTRIPE)],
                        out_hbm.at[pl.ds(c * Q + 15 * QSTRIPE,
                                         Q - 15 * QSTRIPE)])


def _sc_agg(p, lsrc, ldst, cnts, zrows, y):
    kfn = functools.partial(
        pl.kernel,
        out_type=jax.ShapeDtypeStruct((2 * Q, 2, 128), jnp.float32),
        mesh=plsc.VectorSubcoreMesh(core_axis_name="c", subcore_axis_name="s"),
        compiler_params=pltpu.CompilerParams(needs_layout_passes=False),
        scratch_types=[
            pltpu.VMEM((LC, CH), jnp.int32),
            pltpu.VMEM((LC, CH), jnp.int32),
            pltpu.VMEM((8, 128), jnp.int32),
            pltpu.VMEM((CH, 2, 128), jnp.float32),
            pltpu.VMEM((CH, 2, 128), jnp.float32),
            pltpu.SemaphoreType.DMA,
            pltpu.SemaphoreType.DMA,
            pltpu.VMEM_SHARED((ROWS_Q, 2, 128), jnp.float32),
        ],
    )
    return kfn(functools.partial(_agg_body, p))(lsrc, ldst, cnts, zrows, y)


def _b_body(xw_ref, deg_ref, y_ref, dinv_ref):
    dinv = lax.rsqrt(deg_ref[...][:, :1] + 1.0)
    y_ref[...] = xw_ref[...] * dinv
    dinv_ref[...] = dinv


def _tc_b(xw1, deg):
    return pl.pallas_call(
        _b_body,
        grid=(10,),
        in_specs=[
            pl.BlockSpec((1000, 256), lambda i: (i, 0)),
            pl.BlockSpec((1000, 128), lambda i: (i, 0)),
        ],
        out_specs=[
            pl.BlockSpec((1000, 256), lambda i: (i, 0)),
            pl.BlockSpec((1000, 1), lambda i: (i, 0)),
        ],
        out_shape=[
            jax.ShapeDtypeStruct((N, 256), jnp.float32),
            jax.ShapeDtypeStruct((N, 1), jnp.float32),
        ],
    )(xw1, deg)


def _d_body(agg_ref, y_ref, dinv_ref, b_ref, w_ref, y2_ref):
    dinv = dinv_ref[...]
    h1 = jnp.maximum(dinv * (agg_ref[...] + y_ref[...]) + b_ref[...], 0.0)
    y2_ref[...] = jnp.dot(h1, w_ref[...],
                          preferred_element_type=jnp.float32) * dinv


def _tc_d(agg1, y1, dinv, b1, W2):
    return pl.pallas_call(
        _d_body,
        grid=(10,),
        in_specs=[
            pl.BlockSpec((1000, 256), lambda i: (i, 0)),
            pl.BlockSpec((1000, 256), lambda i: (i, 0)),
            pl.BlockSpec((1000, 1), lambda i: (i, 0)),
            pl.BlockSpec((1, 256), lambda i: (0, 0)),
            pl.BlockSpec((256, 256), lambda i: (0, 0)),
        ],
        out_specs=pl.BlockSpec((1000, 256), lambda i: (i, 0)),
        out_shape=jax.ShapeDtypeStruct((N, 256), jnp.float32),
    )(agg1, y1, dinv, b1, W2)


def _f_body(agg_ref, y_ref, dinv_ref, b2_ref, batch_ref,
            wih_ref, bih_ref, bhh_ref, wfc_ref, bfc_ref,
            out_ref, sum_acc, cnt_acc):
    i = pl.program_id(0)

    @pl.when(i == 0)
    def _():
        sum_acc[...] = jnp.zeros_like(sum_acc)
        cnt_acc[...] = jnp.zeros_like(cnt_acc)

    dinv = dinv_ref[...]
    h2 = jnp.maximum(dinv * (agg_ref[...] + y_ref[...]) + b2_ref[...], 0.0)
    b = batch_ref[...]
    gids = lax.broadcasted_iota(jnp.int32, (1000, 64), 1)
    onehot = (b == gids).astype(jnp.float32)
    sum_acc[...] += lax.dot_general(onehot, h2, (((0,), (0,)), ((), ())),
                                    preferred_element_type=jnp.float32)
    cnt_acc[...] += lax.dot_general(onehot, jnp.ones((1000, 1), jnp.float32),
                                    (((0,), (0,)), ((), ())),
                                    preferred_element_type=jnp.float32)

    @pl.when(i == 9)
    def _():
        cnt = jnp.maximum(cnt_acc[...][:, :1], 1.0)
        pooled = sum_acc[...] / cnt
        gi = lax.dot_general(pooled, wih_ref[...], (((1,), (1,)), ((), ())),
                             preferred_element_type=jnp.float32) + bih_ref[...]
        bhh = bhh_ref[...]
        r = jax.nn.sigmoid(gi[:, 0:256] + bhh[:, 0:256])
        z = jax.nn.sigmoid(gi[:, 256:512] + bhh[:, 256:512])
        n = jnp.tanh(gi[:, 512:768] + r * bhh[:, 512:768])
        gru = (1.0 - z) * n
        out_ref[...] = lax.dot_general(gru, wfc_ref[...],
                                       (((1,), (1,)), ((), ())),
                                       preferred_element_type=jnp.float32
                                       ) + bfc_ref[...]


def _tc_f(agg2, y2, dinv, b2, batch2d, W_ih, b_ih2d, b_hh2d, Wfc, bfc2d):
    return pl.pallas_call(
        _f_body,
        grid=(10,),
        in_specs=[
            pl.BlockSpec((1000, 256), lambda i: (i, 0)),
            pl.BlockSpec((1000, 256), lambda i: (i, 0)),
            pl.BlockSpec((1000, 1), lambda i: (i, 0)),
            pl.BlockSpec((1, 256), lambda i: (0, 0)),
            pl.BlockSpec((1000, 1), lambda i: (i, 0)),
            pl.BlockSpec((768, 256), lambda i: (0, 0)),
            pl.BlockSpec((1, 768), lambda i: (0, 0)),
            pl.BlockSpec((1, 768), lambda i: (0, 0)),
            pl.BlockSpec((10, 256), lambda i: (0, 0)),
            pl.BlockSpec((1, 10), lambda i: (0, 0)),
        ],
        out_specs=pl.BlockSpec((64, 10), lambda i: (0, 0)),
        out_shape=jax.ShapeDtypeStruct((64, 10), jnp.float32),
        scratch_shapes=[
            pltpu.VMEM((64, 256), jnp.float32),
            pltpu.VMEM((64, 1), jnp.float32),
        ],
    )(agg2, y2, dinv, b2, batch2d, W_ih, b_ih2d, b_hh2d, Wfc, bfc2d)


@jax.jit
def kernel(x, edge_index, batch, W1, b1, W2, b2, W_ih, W_hh, b_ih, b_hh,
           Wfc, bfc):
    src = edge_index[0]
    dst = edge_index[1]
    pad = EPAD - E
    src2d = jnp.concatenate(
        [src, jnp.zeros((pad,), jnp.int32)]).reshape(NCHUNK, CH)
    dst2d = jnp.concatenate(
        [dst, jnp.full((pad,), -1, jnp.int32)]).reshape(NCHUNK, CH)

    xw1, dloc2d = _tc_z(x, W1, dst2d)

    ones128 = jnp.ones((CH, 128), jnp.float32)
    zrows128 = jnp.zeros((QSTRIPE, 128), jnp.float32)
    zrows256 = jnp.zeros((QSTRIPE, 2, 128), jnp.float32)

    lsrc, ldst, cnts = _sc_compact(src2d, dloc2d)
    deg = _sc_deg(ldst, cnts, ones128, zrows128)[:N]
    y1, dinv = _tc_b(xw1, deg)
    y1r = y1.reshape(N, 2, 128)
    agg1 = jnp.concatenate([
        _sc_agg(0, lsrc, ldst, cnts, zrows256, y1r),
        _sc_agg(1, lsrc, ldst, cnts, zrows256, y1r),
    ]).reshape(2 * NPAD // 2, 256)[:N]
    y2 = _tc_d(agg1, y1, dinv, b1.reshape(1, 256), W2)
    y2r = y2.reshape(N, 2, 128)
    agg2 = jnp.concatenate([
        _sc_agg(0, lsrc, ldst, cnts, zrows256, y2r),
        _sc_agg(1, lsrc, ldst, cnts, zrows256, y2r),
    ]).reshape(2 * NPAD // 2, 256)[:N]
    out = _tc_f(agg2, y2, dinv, b2.reshape(1, 256), batch.reshape(N, 1),
                W_ih, b_ih.reshape(1, 768), b_hh.reshape(1, 768),
                Wfc, bfc.reshape(1, 10))
    return out


# speedup vs baseline: 1.1044x; 1.1044x over previous
import functools

import jax
import jax.numpy as jnp
from jax import lax
from jax.experimental import pallas as pl
from jax.experimental.pallas import tpu as pltpu
from jax.experimental.pallas import tpu_sc as plsc

N = 10000
E = 160000
HALF = 5000
CH = 128
CPT = 80
NCHUNK = 16 * CPT
EPAD = NCHUNK * CH
ROWS_ACC = 5120
DUMMY0 = 5008
STRIPE = ROWS_ACC // 16
CPW = NCHUNK // 32
LC = 48


def _z_body(x_ref, w_ref, dst_ref, xw_ref, dloc_ref):
    i = pl.program_id(0)
    xw_ref[...] = jnp.dot(x_ref[...], w_ref[...],
                          preferred_element_type=jnp.float32)

    @pl.when(i == 0)
    def _():
        d = dst_ref[...]
        dummy = DUMMY0 + (d & 63)
        dloc0 = jnp.where((d >= 0) & (d < HALF), d, dummy)
        dloc1 = jnp.where(d >= HALF, d - HALF, dummy)
        dloc_ref[...] = jnp.stack([dloc0, dloc1])


def _tc_z(x, W1, dst2d):
    return pl.pallas_call(
        _z_body,
        grid=(10,),
        in_specs=[
            pl.BlockSpec((1000, 256), lambda i: (i, 0)),
            pl.BlockSpec((256, 256), lambda i: (0, 0)),
            pl.BlockSpec((NCHUNK, CH), lambda i: (0, 0)),
        ],
        out_specs=[
            pl.BlockSpec((1000, 256), lambda i: (i, 0)),
            pl.BlockSpec((2, NCHUNK, CH), lambda i: (0, 0, 0)),
        ],
        out_shape=[
            jax.ShapeDtypeStruct((N, 256), jnp.float32),
            jax.ShapeDtypeStruct((2, NCHUNK, CH), jnp.int32),
        ],
    )(x, W1, dst2d)


def _compact_body(src_hbm, dloc_hbm, lsrc_hbm, ldst_hbm, cnt_hbm,
                  sstage, dstage, lsrc_v, ldst_v, cbuf):
    c = lax.axis_index("c")
    s = lax.axis_index("s")
    w = s * 2 + c
    pltpu.sync_copy(src_hbm.at[pl.ds(w * CPW, CPW)], sstage)
    lanes = lax.iota(jnp.int32, 16)

    for h in range(2):
        pltpu.sync_copy(dloc_hbm.at[h, pl.ds(w * CPW, CPW)], dstage)

        def chunk(k, off):
            for j in range(CH // 16):
                vd = dstage[k, pl.ds(j * 16, 16)]
                vs = sstage[k, pl.ds(j * 16, 16)]
                m = vd < HALF
                mi = jnp.where(m, 1, 0)
                pos = off + plsc.cumsum(mi) - 1
                plsc.store_scatter(lsrc_v, [pos >> 7, pos & 127], vs, mask=m)
                plsc.store_scatter(ldst_v, [pos >> 7, pos & 127], vd, mask=m)
                off = off + jnp.sum(mi)
            return off

        off = lax.fori_loop(0, CPW, chunk, 0)
        for j in range(CH // 16):
            posp = off + j * 16 + lanes
            plsc.store_scatter(lsrc_v, [posp >> 7, posp & 127],
                               jnp.zeros((16,), jnp.int32))
            plsc.store_scatter(ldst_v, [posp >> 7, posp & 127],
                               DUMMY0 + lanes + (j % 4) * 16)
        nch = (off + CH - 1) // CH
        cbuf[0, pl.ds(0, 16)] = jnp.full((16,), nch, jnp.int32)
        pltpu.sync_copy(lsrc_v, lsrc_hbm.at[h, w])
        pltpu.sync_copy(ldst_v, ldst_hbm.at[h, w])
        pltpu.sync_copy(cbuf, cnt_hbm.at[h, w])


def _sc_compact(src2d, dloc2d):
    kfn = functools.partial(
        pl.kernel,
        out_type=[
            jax.ShapeDtypeStruct((2, 32, LC, CH), jnp.int32),
            jax.ShapeDtypeStruct((2, 32, LC, CH), jnp.int32),
            jax.ShapeDtypeStruct((2, 32, 8, 128), jnp.int32),
        ],
        mesh=plsc.VectorSubcoreMesh(core_axis_name="c", subcore_axis_name="s"),
        compiler_params=pltpu.CompilerParams(needs_layout_passes=False),
        scratch_types=[
            pltpu.VMEM((CPW, CH), jnp.int32),
            pltpu.VMEM((CPW, CH), jnp.int32),
            pltpu.VMEM((LC, CH), jnp.int32),
            pltpu.VMEM((LC, CH), jnp.int32),
            pltpu.VMEM((8, 128), jnp.int32),
        ],
    )
    return kfn(_compact_body)(src2d, dloc2d)


def _deg_body(ldst_hbm, cnt_hbm, ones_hbm, zrows_hbm, deg_hbm,
              dlocb, cbuf, ones_v, sem, acc):
    c = lax.axis_index("c")
    s = lax.axis_index("s")
    pltpu.sync_copy(zrows_hbm, acc.at[pl.ds(s * STRIPE, STRIPE)])
    pltpu.sync_copy(ones_hbm, ones_v)
    plsc.subcore_barrier()

    for w in (2 * s, 2 * s + 1):
        pltpu.sync_copy(ldst_hbm.at[c, w], dlocb)
        pltpu.sync_copy(cnt_hbm.at[c, w], cbuf)
        nch = jnp.max(cbuf[0, pl.ds(0, 16)])

        def chunk(k, carry):
            pltpu.async_copy(ones_v, acc.at[dlocb.at[k]], sem, add=True)
            return carry

        lax.fori_loop(0, nch, chunk, 0)

        def drain(k, carry):
            pltpu.make_async_copy(ones_v, acc.at[dlocb.at[0]], sem).wait()
            return carry

        lax.fori_loop(0, nch, drain, 0)

    plsc.subcore_barrier()

    @pl.when(s < 15)
    def _():
        pltpu.sync_copy(acc.at[pl.ds(s * STRIPE, STRIPE)],
                        deg_hbm.at[pl.ds(c * HALF + s * STRIPE, STRIPE)])

    @pl.when(s == 15)
    def _():
        pltpu.sync_copy(acc.at[pl.ds(15 * STRIPE, HALF - 15 * STRIPE)],
                        deg_hbm.at[pl.ds(c * HALF + 15 * STRIPE,
                                         HALF - 15 * STRIPE)])


def _sc_deg(ldst, cnts, ones128, zrows):
    kfn = functools.partial(
        pl.kernel,
        out_type=jax.ShapeDtypeStruct((N, 128), jnp.float32),
        mesh=plsc.VectorSubcoreMesh(core_axis_name="c", subcore_axis_name="s"),
        compiler_params=pltpu.CompilerParams(needs_layout_passes=False),
        scratch_types=[
            pltpu.VMEM((LC, CH), jnp.int32),
            pltpu.VMEM((8, 128), jnp.int32),
            pltpu.VMEM((CH, 128), jnp.float32),
            pltpu.SemaphoreType.DMA,
            pltpu.VMEM_SHARED((ROWS_ACC, 128), jnp.float32),
        ],
    )
    return kfn(_deg_body)(ldst, cnts, ones128, zrows)


def _agg_body(lsrc_hbm, ldst_hbm, cnt_hbm, zrows_hbm, y_hbm, out_hbm,
              srcb, dlocb, cbuf, rows0, rows1, g0, g1, acc):
    c = lax.axis_index("c")
    s = lax.axis_index("s")
    pltpu.sync_copy(zrows_hbm, acc.at[pl.ds(s * STRIPE, STRIPE)])
    plsc.subcore_barrier()

    bufs = (rows0, rows1)
    gsems = (g0, g1)

    def start_gather(k, b):
        pltpu.make_async_copy(y_hbm.at[srcb.at[k]], bufs[b], gsems[b]).start()

    def wait_gather(b):
        pltpu.make_async_copy(y_hbm.at[srcb.at[0]], bufs[b], gsems[b]).wait()

    def scatter(k, b):
        pltpu.sync_copy(bufs[b], acc.at[dlocb.at[k]], add=True)

    for w in (2 * s, 2 * s + 1):
        pltpu.sync_copy(lsrc_hbm.at[c, w], srcb)
        pltpu.sync_copy(ldst_hbm.at[c, w], dlocb)
        pltpu.sync_copy(cnt_hbm.at[c, w], cbuf)
        nch = jnp.max(cbuf[0, pl.ds(0, 16)])

        @pl.when(nch > 0)
        def _():
            start_gather(0, 0)

        def pair(k2, carry):
            k0 = 2 * k2
            k1 = k0 + 1

            @pl.when(k1 < nch)
            def _():
                start_gather(k1, 1)

            wait_gather(0)
            scatter(k0, 0)

            @pl.when(k0 + 2 < nch)
            def _():
                start_gather(k0 + 2, 0)

            @pl.when(k1 < nch)
            def _():
                wait_gather(1)
                scatter(k1, 1)

            return carry

        lax.fori_loop(0, (nch + 1) // 2, pair, 0)

    plsc.subcore_barrier()

    @pl.when(s < 15)
    def _():
        pltpu.sync_copy(acc.at[pl.ds(s * STRIPE, STRIPE)],
                        out_hbm.at[pl.ds(c * HALF + s * STRIPE, STRIPE)])

    @pl.when(s == 15)
    def _():
        pltpu.sync_copy(acc.at[pl.ds(15 * STRIPE, HALF - 15 * STRIPE)],
                        out_hbm.at[pl.ds(c * HALF + 15 * STRIPE,
                                         HALF - 15 * STRIPE)])


def _sc_agg(lsrc, ldst, cnts, zrows, y_half):
    kfn = functools.partial(
        pl.kernel,
        out_type=jax.ShapeDtypeStruct((N, 128), jnp.float32),
        mesh=plsc.VectorSubcoreMesh(core_axis_name="c", subcore_axis_name="s"),
        compiler_params=pltpu.CompilerParams(needs_layout_passes=False),
        scratch_types=[
            pltpu.VMEM((LC, CH), jnp.int32),
            pltpu.VMEM((LC, CH), jnp.int32),
            pltpu.VMEM((8, 128), jnp.int32),
            pltpu.VMEM((CH, 128), jnp.float32),
            pltpu.VMEM((CH, 128), jnp.float32),
            pltpu.SemaphoreType.DMA,
            pltpu.SemaphoreType.DMA,
            pltpu.VMEM_SHARED((ROWS_ACC, 128), jnp.float32),
        ],
    )
    return kfn(_agg_body)(lsrc, ldst, cnts, zrows, y_half)


def _b_body(xw_ref, deg_ref, ya_ref, yb_ref, dinv_ref):
    dinv = lax.rsqrt(deg_ref[...][:, :1] + 1.0)
    y = xw_ref[...] * dinv
    ya_ref[...] = y[:, 0:128]
    yb_ref[...] = y[:, 128:256]
    dinv_ref[...] = dinv


def _tc_b(xw1, deg):
    return pl.pallas_call(
        _b_body,
        grid=(10,),
        in_specs=[
            pl.BlockSpec((1000, 256), lambda i: (i, 0)),
            pl.BlockSpec((1000, 128), lambda i: (i, 0)),
        ],
        out_specs=[
            pl.BlockSpec((1000, 128), lambda i: (i, 0)),
            pl.BlockSpec((1000, 128), lambda i: (i, 0)),
            pl.BlockSpec((1000, 1), lambda i: (i, 0)),
        ],
        out_shape=[
            jax.ShapeDtypeStruct((N, 128), jnp.float32),
            jax.ShapeDtypeStruct((N, 128), jnp.float32),
            jax.ShapeDtypeStruct((N, 1), jnp.float32),
        ],
    )(xw1, deg)


def _d_body(agga_ref, aggb_ref, ya_ref, yb_ref, dinv_ref, b_ref, w_ref,
            y2a_ref, y2b_ref):
    dinv = dinv_ref[...]
    agg = jnp.concatenate([agga_ref[...], aggb_ref[...]], axis=1)
    y1 = jnp.concatenate([ya_ref[...], yb_ref[...]], axis=1)
    h1 = jnp.maximum(dinv * (agg + y1) + b_ref[...], 0.0)
    y2 = jnp.dot(h1, w_ref[...], preferred_element_type=jnp.float32) * dinv
    y2a_ref[...] = y2[:, 0:128]
    y2b_ref[...] = y2[:, 128:256]


def _tc_d(agg1a, agg1b, y1a, y1b, dinv, b1, W2):
    return pl.pallas_call(
        _d_body,
        grid=(10,),
        in_specs=[
            pl.BlockSpec((1000, 128), lambda i: (i, 0)),
            pl.BlockSpec((1000, 128), lambda i: (i, 0)),
            pl.BlockSpec((1000, 128), lambda i: (i, 0)),
            pl.BlockSpec((1000, 128), lambda i: (i, 0)),
            pl.BlockSpec((1000, 1), lambda i: (i, 0)),
            pl.BlockSpec((1, 256), lambda i: (0, 0)),
            pl.BlockSpec((256, 256), lambda i: (0, 0)),
        ],
        out_specs=[
            pl.BlockSpec((1000, 128), lambda i: (i, 0)),
            pl.BlockSpec((1000, 128), lambda i: (i, 0)),
        ],
        out_shape=[
            jax.ShapeDtypeStruct((N, 128), jnp.float32),
            jax.ShapeDtypeStruct((N, 128), jnp.float32),
        ],
    )(agg1a, agg1b, y1a, y1b, dinv, b1, W2)


def _f_body(agga_ref, aggb_ref, ya_ref, yb_ref, dinv_ref, b2_ref, batch_ref,
            wih_ref, bih_ref, bhh_ref, wfc_ref, bfc_ref,
            out_ref, sum_acc, cnt_acc):
    i = pl.program_id(0)

    @pl.when(i == 0)
    def _():
        sum_acc[...] = jnp.zeros_like(sum_acc)
        cnt_acc[...] = jnp.zeros_like(cnt_acc)

    dinv = dinv_ref[...]
    agg = jnp.concatenate([agga_ref[...], aggb_ref[...]], axis=1)
    y2 = jnp.concatenate([ya_ref[...], yb_ref[...]], axis=1)
    h2 = jnp.maximum(dinv * (agg + y2) + b2_ref[...], 0.0)
    b = batch_ref[...]
    gids = lax.broadcasted_iota(jnp.int32, (1000, 64), 1)
    onehot = (b == gids).astype(jnp.float32)
    sum_acc[...] += lax.dot_general(onehot, h2, (((0,), (0,)), ((), ())),
                                    preferred_element_type=jnp.float32)
    cnt_acc[...] += lax.dot_general(onehot, jnp.ones((1000, 1), jnp.float32),
                                    (((0,), (0,)), ((), ())),
                                    preferred_element_type=jnp.float32)

    @pl.when(i == 9)
    def _():
        cnt = jnp.maximum(cnt_acc[...][:, :1], 1.0)
        pooled = sum_acc[...] / cnt
        gi = lax.dot_general(pooled, wih_ref[...], (((1,), (1,)), ((), ())),
                             preferred_element_type=jnp.float32) + bih_ref[...]
        bhh = bhh_ref[...]
        r = jax.nn.sigmoid(gi[:, 0:256] + bhh[:, 0:256])
        z = jax.nn.sigmoid(gi[:, 256:512] + bhh[:, 256:512])
        n = jnp.tanh(gi[:, 512:768] + r * bhh[:, 512:768])
        gru = (1.0 - z) * n
        out_ref[...] = lax.dot_general(gru, wfc_ref[...],
                                       (((1,), (1,)), ((), ())),
                                       preferred_element_type=jnp.float32
                                       ) + bfc_ref[...]


def _tc_f(agg2a, agg2b, y2a, y2b, dinv, b2, batch2d,
          W_ih, b_ih2d, b_hh2d, Wfc, bfc2d):
    return pl.pallas_call(
        _f_body,
        grid=(10,),
        in_specs=[
            pl.BlockSpec((1000, 128), lambda i: (i, 0)),
            pl.BlockSpec((1000, 128), lambda i: (i, 0)),
            pl.BlockSpec((1000, 128), lambda i: (i, 0)),
            pl.BlockSpec((1000, 128), lambda i: (i, 0)),
            pl.BlockSpec((1000, 1), lambda i: (i, 0)),
            pl.BlockSpec((1, 256), lambda i: (0, 0)),
            pl.BlockSpec((1000, 1), lambda i: (i, 0)),
            pl.BlockSpec((768, 256), lambda i: (0, 0)),
            pl.BlockSpec((1, 768), lambda i: (0, 0)),
            pl.BlockSpec((1, 768), lambda i: (0, 0)),
            pl.BlockSpec((10, 256), lambda i: (0, 0)),
            pl.BlockSpec((1, 10), lambda i: (0, 0)),
        ],
        out_specs=pl.BlockSpec((64, 10), lambda i: (0, 0)),
        out_shape=jax.ShapeDtypeStruct((64, 10), jnp.float32),
        scratch_shapes=[
            pltpu.VMEM((64, 256), jnp.float32),
            pltpu.VMEM((64, 1), jnp.float32),
        ],
    )(agg2a, agg2b, y2a, y2b, dinv, b2, batch2d,
      W_ih, b_ih2d, b_hh2d, Wfc, bfc2d)


@jax.jit
def kernel(x, edge_index, batch, W1, b1, W2, b2, W_ih, W_hh, b_ih, b_hh,
           Wfc, bfc):
    src = edge_index[0]
    dst = edge_index[1]
    pad = EPAD - E
    src2d = jnp.concatenate(
        [src, jnp.zeros((pad,), jnp.int32)]).reshape(NCHUNK, CH)
    dst2d = jnp.concatenate(
        [dst, jnp.full((pad,), -1, jnp.int32)]).reshape(NCHUNK, CH)

    xw1, dloc2d = _tc_z(x, W1, dst2d)

    ones128 = jnp.ones((CH, 128), jnp.float32)
    zrows = jnp.zeros((STRIPE, 128), jnp.float32)

    lsrc, ldst, cnts = _sc_compact(src2d, dloc2d)
    deg = _sc_deg(ldst, cnts, ones128, zrows)
    y1a, y1b, dinv = _tc_b(xw1, deg)
    agg1a = _sc_agg(lsrc, ldst, cnts, zrows, y1a)
    agg1b = _sc_agg(lsrc, ldst, cnts, zrows, y1b)
    y2a, y2b = _tc_d(agg1a, agg1b, y1a, y1b, dinv, b1.reshape(1, 256), W2)
    agg2a = _sc_agg(lsrc, ldst, cnts, zrows, y2a)
    agg2b = _sc_agg(lsrc, ldst, cnts, zrows, y2b)
    out = _tc_f(agg2a, agg2b, y2a, y2b, dinv, b2.reshape(1, 256),
                batch.reshape(N, 1),
                W_ih, b_ih.reshape(1, 768), b_hh.reshape(1, 768),
                Wfc, bfc.reshape(1, 10))
    return out


# speedup vs baseline: 1.1196x; 1.0138x over previous
import functools

import jax
import jax.numpy as jnp
from jax import lax
from jax.experimental import pallas as pl
from jax.experimental.pallas import tpu as pltpu
from jax.experimental.pallas import tpu_sc as plsc

N = 10000
E = 160000
HALF = 5000
CH = 128
CPT = 80
NCHUNK = 16 * CPT
EPAD = NCHUNK * CH
ROWS_ACC = 5120
DUMMY0 = 5008
STRIPE = ROWS_ACC // 16
CPW = NCHUNK // 32
LC = 48


def _z_body(x_ref, w_ref, dst_ref, xw_ref, dloc_ref):
    i = pl.program_id(0)
    xw_ref[...] = jnp.dot(x_ref[...], w_ref[...],
                          preferred_element_type=jnp.float32)

    @pl.when(i == 0)
    def _():
        d = dst_ref[...]
        dummy = DUMMY0 + (d & 63)
        dloc0 = jnp.where((d >= 0) & (d < HALF), d, dummy)
        dloc1 = jnp.where(d >= HALF, d - HALF, dummy)
        dloc_ref[...] = jnp.stack([dloc0, dloc1])


def _tc_z(x, W1, dst2d):
    return pl.pallas_call(
        _z_body,
        grid=(10,),
        in_specs=[
            pl.BlockSpec((1000, 256), lambda i: (i, 0)),
            pl.BlockSpec((256, 256), lambda i: (0, 0)),
            pl.BlockSpec((NCHUNK, CH), lambda i: (0, 0)),
        ],
        out_specs=[
            pl.BlockSpec((1000, 256), lambda i: (i, 0)),
            pl.BlockSpec((2, NCHUNK, CH), lambda i: (0, 0, 0)),
        ],
        out_shape=[
            jax.ShapeDtypeStruct((N, 256), jnp.float32),
            jax.ShapeDtypeStruct((2, NCHUNK, CH), jnp.int32),
        ],
    )(x, W1, dst2d)


def _compact_body(src_hbm, dloc_hbm, lsrc_hbm, ldst_hbm, cnt_hbm,
                  sstage, dstage, lsrc_v, ldst_v, cbuf):
    c = lax.axis_index("c")
    s = lax.axis_index("s")
    w = s * 2 + c
    pltpu.sync_copy(src_hbm.at[pl.ds(w * CPW, CPW)], sstage)
    lanes = lax.iota(jnp.int32, 16)

    for h in range(2):
        pltpu.sync_copy(dloc_hbm.at[h, pl.ds(w * CPW, CPW)], dstage)

        def chunk(k, off):
            for j in range(CH // 16):
                vd = dstage[k, pl.ds(j * 16, 16)]
                vs = sstage[k, pl.ds(j * 16, 16)]
                m = vd < HALF
                mi = jnp.where(m, 1, 0)
                pos = off + plsc.cumsum(mi) - 1
                plsc.store_scatter(lsrc_v, [pos >> 7, pos & 127], vs, mask=m)
                plsc.store_scatter(ldst_v, [pos >> 7, pos & 127], vd, mask=m)
                off = off + jnp.sum(mi)
            return off

        off = lax.fori_loop(0, CPW, chunk, 0)
        for j in range(CH // 16):
            posp = off + j * 16 + lanes
            plsc.store_scatter(lsrc_v, [posp >> 7, posp & 127],
                               jnp.zeros((16,), jnp.int32))
            plsc.store_scatter(ldst_v, [posp >> 7, posp & 127],
                               DUMMY0 + lanes + (j % 4) * 16)
        nch = (off + CH - 1) // CH
        cbuf[0, pl.ds(0, 16)] = jnp.full((16,), nch, jnp.int32)
        pltpu.sync_copy(lsrc_v, lsrc_hbm.at[h, w])
        pltpu.sync_copy(ldst_v, ldst_hbm.at[h, w])
        pltpu.sync_copy(cbuf, cnt_hbm.at[h, w])


def _sc_compact(src2d, dloc2d):
    kfn = functools.partial(
        pl.kernel,
        out_type=[
            jax.ShapeDtypeStruct((2, 32, LC, CH), jnp.int32),
            jax.ShapeDtypeStruct((2, 32, LC, CH), jnp.int32),
            jax.ShapeDtypeStruct((2, 32, 8, 128), jnp.int32),
        ],
        mesh=plsc.VectorSubcoreMesh(core_axis_name="c", subcore_axis_name="s"),
        compiler_params=pltpu.CompilerParams(needs_layout_passes=False),
        scratch_types=[
            pltpu.VMEM((CPW, CH), jnp.int32),
            pltpu.VMEM((CPW, CH), jnp.int32),
            pltpu.VMEM((LC, CH), jnp.int32),
            pltpu.VMEM((LC, CH), jnp.int32),
            pltpu.VMEM((8, 128), jnp.int32),
        ],
    )
    return kfn(_compact_body)(src2d, dloc2d)


def _deg_body(ldst_hbm, cnt_hbm, ones_hbm, zrows_hbm, deg_hbm,
              dlocb, cbuf, ones_v, sem, acc):
    c = lax.axis_index("c")
    s = lax.axis_index("s")
    pltpu.sync_copy(zrows_hbm, acc.at[pl.ds(s * STRIPE, STRIPE)])
    pltpu.sync_copy(ones_hbm, ones_v)
    plsc.subcore_barrier()

    for w in (2 * s, 2 * s + 1):
        pltpu.sync_copy(ldst_hbm.at[c, w], dlocb)
        pltpu.sync_copy(cnt_hbm.at[c, w], cbuf)
        nch = jnp.max(cbuf[0, pl.ds(0, 16)])

        def chunk(k, carry):
            pltpu.async_copy(ones_v, acc.at[dlocb.at[k]], sem, add=True)
            return carry

        lax.fori_loop(0, nch, chunk, 0)

        def drain(k, carry):
            pltpu.make_async_copy(ones_v, acc.at[dlocb.at[0]], sem).wait()
            return carry

        lax.fori_loop(0, nch, drain, 0)

    plsc.subcore_barrier()

    @pl.when(s < 15)
    def _():
        pltpu.sync_copy(acc.at[pl.ds(s * STRIPE, STRIPE)],
                        deg_hbm.at[pl.ds(c * HALF + s * STRIPE, STRIPE)])

    @pl.when(s == 15)
    def _():
        pltpu.sync_copy(acc.at[pl.ds(15 * STRIPE, HALF - 15 * STRIPE)],
                        deg_hbm.at[pl.ds(c * HALF + 15 * STRIPE,
                                         HALF - 15 * STRIPE)])


def _sc_deg(ldst, cnts, ones128, zrows):
    kfn = functools.partial(
        pl.kernel,
        out_type=jax.ShapeDtypeStruct((N, 128), jnp.float32),
        mesh=plsc.VectorSubcoreMesh(core_axis_name="c", subcore_axis_name="s"),
        compiler_params=pltpu.CompilerParams(needs_layout_passes=False),
        scratch_types=[
            pltpu.VMEM((LC, CH), jnp.int32),
            pltpu.VMEM((8, 128), jnp.int32),
            pltpu.VMEM((CH, 128), jnp.float32),
            pltpu.SemaphoreType.DMA,
            pltpu.VMEM_SHARED((ROWS_ACC, 128), jnp.float32),
        ],
    )
    return kfn(_deg_body)(ldst, cnts, ones128, zrows)


def _agg_body(lsrc_hbm, ldst_hbm, cnt_hbm, zrows_hbm, ya_hbm, yb_hbm,
              outa_hbm, outb_hbm,
              srcb0, dlocb0, srcb1, dlocb1, cbuf, rows0, rows1, g0, g1, acc):
    c = lax.axis_index("c")
    s = lax.axis_index("s")
    bufs = (rows0, rows1)
    gsems = (g0, g1)
    lists = ((srcb0, dlocb0), (srcb1, dlocb1))

    nchs = []
    for i, w in enumerate((2 * s, 2 * s + 1)):
        pltpu.sync_copy(lsrc_hbm.at[c, w], lists[i][0])
        pltpu.sync_copy(ldst_hbm.at[c, w], lists[i][1])
        pltpu.sync_copy(cnt_hbm.at[c, w], cbuf)
        nchs.append(jnp.max(cbuf[0, pl.ds(0, 16)]))

    for y_hbm, out_hbm in ((ya_hbm, outa_hbm), (yb_hbm, outb_hbm)):
        pltpu.sync_copy(zrows_hbm, acc.at[pl.ds(s * STRIPE, STRIPE)])
        plsc.subcore_barrier()

        for i in range(2):
            srcb, dlocb = lists[i]
            nch = nchs[i]

            def start_gather(k, b, srcb=srcb, y_hbm=y_hbm):
                pltpu.make_async_copy(y_hbm.at[srcb.at[k]], bufs[b],
                                      gsems[b]).start()

            def wait_gather(b, srcb=srcb, y_hbm=y_hbm):
                pltpu.make_async_copy(y_hbm.at[srcb.at[0]], bufs[b],
                                      gsems[b]).wait()

            def scatter(k, b, dlocb=dlocb):
                pltpu.sync_copy(bufs[b], acc.at[dlocb.at[k]], add=True)

            @pl.when(nch > 0)
            def _():
                start_gather(0, 0)

            def pair(k2, carry):
                k0 = 2 * k2
                k1 = k0 + 1

                @pl.when(k1 < nch)
                def _():
                    start_gather(k1, 1)

                wait_gather(0)
                scatter(k0, 0)

                @pl.when(k0 + 2 < nch)
                def _():
                    start_gather(k0 + 2, 0)

                @pl.when(k1 < nch)
                def _():
                    wait_gather(1)
                    scatter(k1, 1)

                return carry

            lax.fori_loop(0, (nch + 1) // 2, pair, 0)

        plsc.subcore_barrier()

        @pl.when(s < 15)
        def _(out_hbm=out_hbm):
            pltpu.sync_copy(acc.at[pl.ds(s * STRIPE, STRIPE)],
                            out_hbm.at[pl.ds(c * HALF + s * STRIPE, STRIPE)])

        @pl.when(s == 15)
        def _(out_hbm=out_hbm):
            pltpu.sync_copy(acc.at[pl.ds(15 * STRIPE, HALF - 15 * STRIPE)],
                            out_hbm.at[pl.ds(c * HALF + 15 * STRIPE,
                                             HALF - 15 * STRIPE)])
        plsc.subcore_barrier()


def _sc_agg(lsrc, ldst, cnts, zrows, ya, yb):
    kfn = functools.partial(
        pl.kernel,
        out_type=[
            jax.ShapeDtypeStruct((N, 128), jnp.float32),
            jax.ShapeDtypeStruct((N, 128), jnp.float32),
        ],
        mesh=plsc.VectorSubcoreMesh(core_axis_name="c", subcore_axis_name="s"),
        compiler_params=pltpu.CompilerParams(needs_layout_passes=False),
        scratch_types=[
            pltpu.VMEM((LC, CH), jnp.int32),
            pltpu.VMEM((LC, CH), jnp.int32),
            pltpu.VMEM((LC, CH), jnp.int32),
            pltpu.VMEM((LC, CH), jnp.int32),
            pltpu.VMEM((8, 128), jnp.int32),
            pltpu.VMEM((CH, 128), jnp.float32),
            pltpu.VMEM((CH, 128), jnp.float32),
            pltpu.SemaphoreType.DMA,
            pltpu.SemaphoreType.DMA,
            pltpu.VMEM_SHARED((ROWS_ACC, 128), jnp.float32),
        ],
    )
    return kfn(_agg_body)(lsrc, ldst, cnts, zrows, ya, yb)


def _b_body(xw_ref, deg_ref, ya_ref, yb_ref, dinv_ref):
    dinv = lax.rsqrt(deg_ref[...][:, :1] + 1.0)
    y = xw_ref[...] * dinv
    ya_ref[...] = y[:, 0:128]
    yb_ref[...] = y[:, 128:256]
    dinv_ref[...] = dinv


def _tc_b(xw1, deg):
    return pl.pallas_call(
        _b_body,
        grid=(10,),
        in_specs=[
            pl.BlockSpec((1000, 256), lambda i: (i, 0)),
            pl.BlockSpec((1000, 128), lambda i: (i, 0)),
        ],
        out_specs=[
            pl.BlockSpec((1000, 128), lambda i: (i, 0)),
            pl.BlockSpec((1000, 128), lambda i: (i, 0)),
            pl.BlockSpec((1000, 1), lambda i: (i, 0)),
        ],
        out_shape=[
            jax.ShapeDtypeStruct((N, 128), jnp.float32),
            jax.ShapeDtypeStruct((N, 128), jnp.float32),
            jax.ShapeDtypeStruct((N, 1), jnp.float32),
        ],
    )(xw1, deg)


def _d_body(agga_ref, aggb_ref, ya_ref, yb_ref, dinv_ref, b_ref, w_ref,
            y2a_ref, y2b_ref):
    dinv = dinv_ref[...]
    agg = jnp.concatenate([agga_ref[...], aggb_ref[...]], axis=1)
    y1 = jnp.concatenate([ya_ref[...], yb_ref[...]], axis=1)
    h1 = jnp.maximum(dinv * (agg + y1) + b_ref[...], 0.0)
    y2 = jnp.dot(h1, w_ref[...], preferred_element_type=jnp.float32) * dinv
    y2a_ref[...] = y2[:, 0:128]
    y2b_ref[...] = y2[:, 128:256]


def _tc_d(agg1a, agg1b, y1a, y1b, dinv, b1, W2):
    return pl.pallas_call(
        _d_body,
        grid=(10,),
        in_specs=[
            pl.BlockSpec((1000, 128), lambda i: (i, 0)),
            pl.BlockSpec((1000, 128), lambda i: (i, 0)),
            pl.BlockSpec((1000, 128), lambda i: (i, 0)),
            pl.BlockSpec((1000, 128), lambda i: (i, 0)),
            pl.BlockSpec((1000, 1), lambda i: (i, 0)),
            pl.BlockSpec((1, 256), lambda i: (0, 0)),
            pl.BlockSpec((256, 256), lambda i: (0, 0)),
        ],
        out_specs=[
            pl.BlockSpec((1000, 128), lambda i: (i, 0)),
            pl.BlockSpec((1000, 128), lambda i: (i, 0)),
        ],
        out_shape=[
            jax.ShapeDtypeStruct((N, 128), jnp.float32),
            jax.ShapeDtypeStruct((N, 128), jnp.float32),
        ],
    )(agg1a, agg1b, y1a, y1b, dinv, b1, W2)


def _f_body(agga_ref, aggb_ref, ya_ref, yb_ref, dinv_ref, b2_ref, batch_ref,
            wih_ref, bih_ref, bhh_ref, wfc_ref, bfc_ref,
            out_ref, sum_acc, cnt_acc):
    i = pl.program_id(0)

    @pl.when(i == 0)
    def _():
        sum_acc[...] = jnp.zeros_like(sum_acc)
        cnt_acc[...] = jnp.zeros_like(cnt_acc)

    dinv = dinv_ref[...]
    agg = jnp.concatenate([agga_ref[...], aggb_ref[...]], axis=1)
    y2 = jnp.concatenate([ya_ref[...], yb_ref[...]], axis=1)
    h2 = jnp.maximum(dinv * (agg + y2) + b2_ref[...], 0.0)
    b = batch_ref[...]
    gids = lax.broadcasted_iota(jnp.int32, (1000, 64), 1)
    onehot = (b == gids).astype(jnp.float32)
    sum_acc[...] += lax.dot_general(onehot, h2, (((0,), (0,)), ((), ())),
                                    preferred_element_type=jnp.float32)
    cnt_acc[...] += lax.dot_general(onehot, jnp.ones((1000, 1), jnp.float32),
                                    (((0,), (0,)), ((), ())),
                                    preferred_element_type=jnp.float32)

    @pl.when(i == 9)
    def _():
        cnt = jnp.maximum(cnt_acc[...][:, :1], 1.0)
        pooled = sum_acc[...] / cnt
        gi = lax.dot_general(pooled, wih_ref[...], (((1,), (1,)), ((), ())),
                             preferred_element_type=jnp.float32) + bih_ref[...]
        bhh = bhh_ref[...]
        r = jax.nn.sigmoid(gi[:, 0:256] + bhh[:, 0:256])
        z = jax.nn.sigmoid(gi[:, 256:512] + bhh[:, 256:512])
        n = jnp.tanh(gi[:, 512:768] + r * bhh[:, 512:768])
        gru = (1.0 - z) * n
        out_ref[...] = lax.dot_general(gru, wfc_ref[...],
                                       (((1,), (1,)), ((), ())),
                                       preferred_element_type=jnp.float32
                                       ) + bfc_ref[...]


def _tc_f(agg2a, agg2b, y2a, y2b, dinv, b2, batch2d,
          W_ih, b_ih2d, b_hh2d, Wfc, bfc2d):
    return pl.pallas_call(
        _f_body,
        grid=(10,),
        in_specs=[
            pl.BlockSpec((1000, 128), lambda i: (i, 0)),
            pl.BlockSpec((1000, 128), lambda i: (i, 0)),
            pl.BlockSpec((1000, 128), lambda i: (i, 0)),
            pl.BlockSpec((1000, 128), lambda i: (i, 0)),
            pl.BlockSpec((1000, 1), lambda i: (i, 0)),
            pl.BlockSpec((1, 256), lambda i: (0, 0)),
            pl.BlockSpec((1000, 1), lambda i: (i, 0)),
            pl.BlockSpec((768, 256), lambda i: (0, 0)),
            pl.BlockSpec((1, 768), lambda i: (0, 0)),
            pl.BlockSpec((1, 768), lambda i: (0, 0)),
            pl.BlockSpec((10, 256), lambda i: (0, 0)),
            pl.BlockSpec((1, 10), lambda i: (0, 0)),
        ],
        out_specs=pl.BlockSpec((64, 10), lambda i: (0, 0)),
        out_shape=jax.ShapeDtypeStruct((64, 10), jnp.float32),
        scratch_shapes=[
            pltpu.VMEM((64, 256), jnp.float32),
            pltpu.VMEM((64, 1), jnp.float32),
        ],
    )(agg2a, agg2b, y2a, y2b, dinv, b2, batch2d,
      W_ih, b_ih2d, b_hh2d, Wfc, bfc2d)


@jax.jit
def kernel(x, edge_index, batch, W1, b1, W2, b2, W_ih, W_hh, b_ih, b_hh,
           Wfc, bfc):
    src = edge_index[0]
    dst = edge_index[1]
    pad = EPAD - E
    src2d = jnp.concatenate(
        [src, jnp.zeros((pad,), jnp.int32)]).reshape(NCHUNK, CH)
    dst2d = jnp.concatenate(
        [dst, jnp.full((pad,), -1, jnp.int32)]).reshape(NCHUNK, CH)

    xw1, dloc2d = _tc_z(x, W1, dst2d)

    ones128 = jnp.ones((CH, 128), jnp.float32)
    zrows = jnp.zeros((STRIPE, 128), jnp.float32)

    lsrc, ldst, cnts = _sc_compact(src2d, dloc2d)
    deg = _sc_deg(ldst, cnts, ones128, zrows)
    y1a, y1b, dinv = _tc_b(xw1, deg)
    agg1a, agg1b = _sc_agg(lsrc, ldst, cnts, zrows, y1a, y1b)
    y2a, y2b = _tc_d(agg1a, agg1b, y1a, y1b, dinv, b1.reshape(1, 256), W2)
    agg2a, agg2b = _sc_agg(lsrc, ldst, cnts, zrows, y2a, y2b)
    out = _tc_f(agg2a, agg2b, y2a, y2b, dinv, b2.reshape(1, 256),
                batch.reshape(N, 1),
                W_ih, b_ih.reshape(1, 768), b_hh.reshape(1, 768),
                Wfc, bfc.reshape(1, 10))
    return out
